# baseline (speedup 1.0000x reference)
import functools

import jax
import jax.numpy as jnp
from jax import lax
from jax.experimental import pallas as pl
from jax.experimental.pallas import tpu as pltpu
from jax.experimental.pallas import tpu_sc as plsc

EMBED_DIM = 64
BS = 4096
NUM_AXIOMS = 200
VOCAB2 = 500000
N_PAIRS = BS * NUM_AXIOMS
N_ENT = 2 * N_PAIRS
NW = 32
ENT_PER_W = N_ENT // NW
PAIRS_PER_W = N_PAIRS // NW
CHUNK_ENT = 256
CHUNK_PAIRS = CHUNK_ENT // 2
N_CHUNKS = ENT_PER_W // CHUNK_ENT
IDX_MINOR = 128
SUB = CHUNK_ENT // IDX_MINOR
IDX_ROWS_PER_W = ENT_PER_W // IDX_MINOR
N_GROUPS = CHUNK_PAIRS // 16


def _sc_score(x2d, emb2):
    mesh = plsc.VectorSubcoreMesh(core_axis_name="c", subcore_axis_name="s")

    @functools.partial(
        pl.kernel,
        mesh=mesh,
        out_type=jax.ShapeDtypeStruct((N_PAIRS,), jnp.float32),
        compiler_params=pltpu.CompilerParams(needs_layout_passes=False),
        scratch_types=[
            pltpu.VMEM((IDX_ROWS_PER_W, IDX_MINOR), jnp.int32),
            pltpu.VMEM((CHUNK_ENT, 2 * EMBED_DIM), jnp.float32),
            pltpu.VMEM((CHUNK_ENT, 2 * EMBED_DIM), jnp.float32),
            pltpu.VMEM((SUB, IDX_MINOR), jnp.int32),
            pltpu.VMEM((SUB, IDX_MINOR), jnp.int32),
            pltpu.VMEM((CHUNK_PAIRS,), jnp.float32),
            pltpu.SemaphoreType.DMA,
            pltpu.SemaphoreType.DMA,
        ],
    )
    def k(x_hbm, emb_hbm, out_hbm, idx_v, rows0, rows1, sup0, sup1, out_v,
          sem0, sem1):
        wid = lax.axis_index("s") * 2 + lax.axis_index("c")
        pltpu.sync_copy(
            x_hbm.at[pl.ds(wid * IDX_ROWS_PER_W, IDX_ROWS_PER_W)], idx_v)

        def fire(g, rows, sup, sem):
            for j in range(SUB):
                for kk in range(IDX_MINOR // 16):
                    v = idx_v[g * SUB + j, pl.ds(kk * 16, 16)]
                    sup[j, pl.ds(kk * 16, 16)] = lax.shift_right_logical(v, 1)
            for j in range(SUB):
                pltpu.async_copy(
                    emb_hbm.at[sup.at[j]],
                    rows.at[pl.ds(j * IDX_MINOR, IDX_MINOR)],
                    sem)

        def drain(rows, sem):
            pltpu.make_async_copy(
                emb_hbm.at[pl.ds(0, CHUNK_ENT)], rows, sem).wait()

        lanes = lax.iota(jnp.int32, 16)

        def compute(g, rows):
            def group(t, carry):
                e_l = 32 * t + 2 * lanes
                e_r = e_l + 1
                irow = g * SUB + lax.shift_right_logical(e_l, 7)
                icol_l = lax.bitwise_and(e_l, 127)
                raw_l = plsc.load_gather(idx_v, [irow, icol_l])
                raw_r = plsc.load_gather(idx_v, [irow, icol_l + 1])
                half_l = lax.bitwise_and(raw_l, 1) * EMBED_DIM
                half_r = lax.bitwise_and(raw_r, 1) * EMBED_DIM
                base_col = lanes + t
                acc = jnp.zeros((16,), jnp.float32)
                for d in range(EMBED_DIM):
                    rot = lax.bitwise_and(base_col + d, 63)
                    vl = plsc.load_gather(rows, [e_l, half_l + rot])
                    vr = plsc.load_gather(rows, [e_r, half_r + rot])
                    acc = acc + vl * vr
                out_v[pl.ds(t * 16, 16)] = acc
                return carry
            lax.fori_loop(0, N_GROUPS, group, 0)
            out_base = wid * PAIRS_PER_W + g * CHUNK_PAIRS
            pltpu.sync_copy(out_v, out_hbm.at[pl.ds(out_base, CHUNK_PAIRS)])

        fire(0, rows0, sup0, sem0)

        def body(i, carry):
            g0 = 2 * i
            fire(g0 + 1, rows1, sup1, sem1)
            drain(rows0, sem0)
            compute(g0, rows0)

            @pl.when(i < N_CHUNKS // 2 - 1)
            def _():
                fire(g0 + 2, rows0, sup0, sem0)

            drain(rows1, sem1)
            compute(g0 + 1, rows1)
            return carry

        lax.fori_loop(0, N_CHUNKS // 2, body, 0)

    return k(x2d, emb2)


def kernel(x, emb):
    x2d = x.astype(jnp.int32).reshape(N_ENT // IDX_MINOR, IDX_MINOR)
    emb2 = emb.reshape(VOCAB2, 2 * EMBED_DIM)
    scores = _sc_score(x2d, emb2)
    return scores.reshape(BS, NUM_AXIOMS)


# speedup vs baseline: 1.0371x; 1.0371x over previous
import functools

import jax
import jax.numpy as jnp
from jax import lax
from jax.experimental import pallas as pl
from jax.experimental.pallas import tpu as pltpu
from jax.experimental.pallas import tpu_sc as plsc

EMBED_DIM = 64
BS = 4096
NUM_AXIOMS = 200
VOCAB2 = 500000
N_PAIRS = BS * NUM_AXIOMS
N_ENT = 2 * N_PAIRS
NW = 32
ENT_PER_W = N_ENT // NW
PAIRS_PER_W = N_PAIRS // NW
CHUNK_ENT = 256
CHUNK_PAIRS = CHUNK_ENT // 2
N_CHUNKS = ENT_PER_W // CHUNK_ENT
IDX_MINOR = 128
SUB = CHUNK_ENT // IDX_MINOR
IDX_ROWS_PER_W = ENT_PER_W // IDX_MINOR
N_GROUPS = CHUNK_PAIRS // 16


RETILE_BLOCK = 160
N_BLOCKS = VOCAB2 // RETILE_BLOCK
BLOCKS_PER_W = 2 * (-(-N_BLOCKS // NW) // 2 + 1)


def _sc_retile(emb):
    mesh = plsc.VectorSubcoreMesh(core_axis_name="c", subcore_axis_name="s")

    @functools.partial(
        pl.kernel,
        mesh=mesh,
        out_type=jax.ShapeDtypeStruct((VOCAB2, 2 * EMBED_DIM), jnp.float32),
        compiler_params=pltpu.CompilerParams(needs_layout_passes=False),
        scratch_types=[
            pltpu.VMEM((2 * RETILE_BLOCK, EMBED_DIM), jnp.float32),
            pltpu.VMEM((2 * RETILE_BLOCK, EMBED_DIM), jnp.float32),
            pltpu.VMEM((RETILE_BLOCK, 2 * EMBED_DIM), jnp.float32),
            pltpu.SemaphoreType.DMA,
            pltpu.SemaphoreType.DMA,
        ],
    )
    def k(emb_hbm, out_hbm, buf0, buf1, buf2, sem0, sem1):
        wid = lax.axis_index("s") * 2 + lax.axis_index("c")

        def blk_of(s):
            b = wid + NW * s
            return jnp.where(b < N_BLOCKS, b, wid)

        def fire(s, buf, sem):
            pltpu.async_copy(
                emb_hbm.at[pl.ds(blk_of(s) * 2 * RETILE_BLOCK,
                                 2 * RETILE_BLOCK)],
                buf, sem)

        def drain(buf, sem):
            pltpu.make_async_copy(
                emb_hbm.at[pl.ds(0, 2 * RETILE_BLOCK)], buf, sem).wait()

        def put(s, buf):
            def pack(j, carry):
                for u in range(2):
                    for kk in range(EMBED_DIM // 16):
                        buf2[j, pl.ds(u * EMBED_DIM + kk * 16, 16)] = (
                            buf[2 * j + u, pl.ds(kk * 16, 16)])
                return carry
            lax.fori_loop(0, RETILE_BLOCK, pack, 0)
            pltpu.sync_copy(
                buf2,
                out_hbm.at[pl.ds(blk_of(s) * RETILE_BLOCK, RETILE_BLOCK)])

        fire(0, buf0, sem0)

        def body(i, carry):
            fire(2 * i + 1, buf1, sem1)
            drain(buf0, sem0)
            put(2 * i, buf0)

            @pl.when(i < BLOCKS_PER_W // 2 - 1)
            def _():
                fire(2 * i + 2, buf0, sem0)

            drain(buf1, sem1)
            put(2 * i + 1, buf1)
            return carry

        lax.fori_loop(0, BLOCKS_PER_W // 2, body, 0)

    return k(emb)


def _sc_score(x2d, emb2):
    mesh = plsc.VectorSubcoreMesh(core_axis_name="c", subcore_axis_name="s")

    @functools.partial(
        pl.kernel,
        mesh=mesh,
        out_type=jax.ShapeDtypeStruct((N_PAIRS,), jnp.float32),
        compiler_params=pltpu.CompilerParams(needs_layout_passes=False),
        scratch_types=[
            pltpu.VMEM((IDX_ROWS_PER_W, IDX_MINOR), jnp.int32),
            pltpu.VMEM((CHUNK_ENT, 2 * EMBED_DIM), jnp.float32),
            pltpu.VMEM((CHUNK_ENT, 2 * EMBED_DIM), jnp.float32),
            pltpu.VMEM((SUB, IDX_MINOR), jnp.int32),
            pltpu.VMEM((SUB, IDX_MINOR), jnp.int32),
            pltpu.VMEM((CHUNK_PAIRS,), jnp.float32),
            pltpu.SemaphoreType.DMA,
            pltpu.SemaphoreType.DMA,
        ],
    )
    def k(x_hbm, emb_hbm, out_hbm, idx_v, rows0, rows1, sup0, sup1, out_v,
          sem0, sem1):
        wid = lax.axis_index("s") * 2 + lax.axis_index("c")
        pltpu.sync_copy(
            x_hbm.at[pl.ds(wid * IDX_ROWS_PER_W, IDX_ROWS_PER_W)], idx_v)

        def fire(g, rows, sup, sem):
            for j in range(SUB):
                for kk in range(IDX_MINOR // 16):
                    v = idx_v[g * SUB + j, pl.ds(kk * 16, 16)]
                    sup[j, pl.ds(kk * 16, 16)] = lax.shift_right_logical(v, 1)
            for j in range(SUB):
                pltpu.async_copy(
                    emb_hbm.at[sup.at[j]],
                    rows.at[pl.ds(j * IDX_MINOR, IDX_MINOR)],
                    sem)

        def drain(rows, sem):
            pltpu.make_async_copy(
                emb_hbm.at[pl.ds(0, CHUNK_ENT)], rows, sem).wait()

        lanes = lax.iota(jnp.int32, 16)

        def compute(g, rows):
            def group(t, carry):
                e_l = 32 * t + 2 * lanes
                e_r = e_l + 1
                irow = g * SUB + lax.shift_right_logical(e_l, 7)
                icol_l = lax.bitwise_and(e_l, 127)
                raw_l = plsc.load_gather(idx_v, [irow, icol_l])
                raw_r = plsc.load_gather(idx_v, [irow, icol_l + 1])
                half_l = lax.bitwise_and(raw_l, 1) * EMBED_DIM
                half_r = lax.bitwise_and(raw_r, 1) * EMBED_DIM
                base_col = lanes + t
                acc = jnp.zeros((16,), jnp.float32)
                for d in range(EMBED_DIM):
                    rot = lax.bitwise_and(base_col + d, 63)
                    vl = plsc.load_gather(rows, [e_l, half_l + rot])
                    vr = plsc.load_gather(rows, [e_r, half_r + rot])
                    acc = acc + vl * vr
                out_v[pl.ds(t * 16, 16)] = acc
                return carry
            lax.fori_loop(0, N_GROUPS, group, 0)
            out_base = wid * PAIRS_PER_W + g * CHUNK_PAIRS
            pltpu.sync_copy(out_v, out_hbm.at[pl.ds(out_base, CHUNK_PAIRS)])

        fire(0, rows0, sup0, sem0)

        def body(i, carry):
            g0 = 2 * i
            fire(g0 + 1, rows1, sup1, sem1)
            drain(rows0, sem0)
            compute(g0, rows0)

            @pl.when(i < N_CHUNKS // 2 - 1)
            def _():
                fire(g0 + 2, rows0, sup0, sem0)

            drain(rows1, sem1)
            compute(g0 + 1, rows1)
            return carry

        lax.fori_loop(0, N_CHUNKS // 2, body, 0)

    return k(x2d, emb2)


def kernel(x, emb):
    x2d = x.astype(jnp.int32).reshape(N_ENT // IDX_MINOR, IDX_MINOR)
    emb2 = _sc_retile(emb)
    scores = _sc_score(x2d, emb2)
    return scores.reshape(BS, NUM_AXIOMS)


# speedup vs baseline: 1.3129x; 1.2660x over previous
import functools

import jax
import jax.numpy as jnp
from jax import lax
from jax.experimental import pallas as pl
from jax.experimental.pallas import tpu as pltpu
from jax.experimental.pallas import tpu_sc as plsc

EMBED_DIM = 64
BS = 4096
NUM_AXIOMS = 200
VOCAB = 1000000
VOCAB2 = VOCAB // 2
NW = 32
STRIPE = BS // NW
X2_ROWS = NUM_AXIOMS * NW * 2

RT_COLS = 512
RT_SUP = RT_COLS // 2
RT_NBLK = VOCAB // RT_COLS
RT_SLOTS = 2 * (-(-RT_NBLK // NW) // 2 + 1)
RT_TAIL_COL = RT_NBLK * RT_COLS
RT_TAIL_N = VOCAB - RT_TAIL_COL


def _sc_retile(embT, tail16k):
    mesh = plsc.VectorSubcoreMesh(core_axis_name="c", subcore_axis_name="s")

    @functools.partial(
        pl.kernel,
        mesh=mesh,
        out_type=jax.ShapeDtypeStruct((VOCAB2, 2 * EMBED_DIM), jnp.float32),
        compiler_params=pltpu.CompilerParams(needs_layout_passes=False),
        scratch_types=[
            pltpu.VMEM((EMBED_DIM, RT_COLS), jnp.float32),
            pltpu.VMEM((EMBED_DIM, RT_COLS), jnp.float32),
            pltpu.VMEM((RT_SUP, 2 * EMBED_DIM), jnp.float32),
            pltpu.SemaphoreType.DMA,
            pltpu.SemaphoreType.DMA,
        ],
    )
    def k(embT_hbm, tail_hbm, out_hbm, buf0, buf1, tbuf, sem0, sem1):
        wid = lax.axis_index("s") * 2 + lax.axis_index("c")
        lanes = lax.iota(jnp.int32, 16)
        half64 = lax.bitwise_and(lanes, 1) * EMBED_DIM

        def blk_of(s):
            return jnp.minimum(wid + NW * s, RT_NBLK - 1)

        def fire(s, buf, sem):
            pltpu.async_copy(
                embT_hbm.at[:, pl.ds(blk_of(s) * RT_COLS, RT_COLS)], buf, sem)

        def drain(buf, sem):
            pltpu.make_async_copy(
                embT_hbm.at[:, pl.ds(0, RT_COLS)], buf, sem).wait()

        def transpose(buf, n_colgroups):
            def colgroup(cg, carry):
                c0 = cg * 16
                cols = c0 + lanes
                srow = lax.shift_right_logical(cols, 1)
                for d0 in range(0, EMBED_DIM, 16):
                    for j in range(16):
                        dsel = d0 + lax.bitwise_and(lanes + j, 15)
                        v = plsc.load_gather(buf, [dsel, cols])
                        plsc.store_scatter(tbuf, [srow, half64 + dsel], v)
                return carry
            lax.fori_loop(0, n_colgroups, colgroup, 0)

        def put(s, buf):
            transpose(buf, RT_COLS // 16)
            pltpu.sync_copy(
                tbuf, out_hbm.at[pl.ds(blk_of(s) * RT_SUP, RT_SUP)])

        fire(0, buf0, sem0)

        def body(i, carry):
            fire(2 * i + 1, buf1, sem1)
            drain(buf0, sem0)
            put(2 * i, buf0)

            @pl.when(i < RT_SLOTS // 2 - 1)
            def _():
                fire(2 * i + 2, buf0, sem0)

            drain(buf1, sem1)
            put(2 * i + 1, buf1)
            return carry

        lax.fori_loop(0, RT_SLOTS // 2, body, 0)

        @pl.when(wid == 0)
        def _():
            pltpu.sync_copy(tail_hbm, tbuf.at[pl.ds(0, RT_TAIL_N // 2)])
            pltpu.sync_copy(
                tbuf.at[pl.ds(0, RT_TAIL_N // 2)],
                out_hbm.at[pl.ds(RT_TAIL_COL // 2, RT_TAIL_N // 2)])

    return k(embT, tail16k)


def _sc_score(x2, emb2):
    mesh = plsc.VectorSubcoreMesh(core_axis_name="c", subcore_axis_name="s")

    @functools.partial(
        pl.kernel,
        mesh=mesh,
        out_type=jax.ShapeDtypeStruct((NUM_AXIOMS, BS), jnp.float32),
        compiler_params=pltpu.CompilerParams(needs_layout_passes=False),
        scratch_types=[
            pltpu.VMEM((2, STRIPE), jnp.int32),
            pltpu.VMEM((2, STRIPE), jnp.int32),
            pltpu.VMEM((STRIPE,), jnp.int32),
            pltpu.VMEM((STRIPE,), jnp.int32),
            pltpu.VMEM((STRIPE,), jnp.int32),
            pltpu.VMEM((STRIPE,), jnp.int32),
            pltpu.VMEM((STRIPE,), jnp.int32),
            pltpu.VMEM((STRIPE,), jnp.int32),
            pltpu.VMEM((STRIPE,), jnp.int32),
            pltpu.VMEM((STRIPE,), jnp.int32),
            pltpu.VMEM((STRIPE, 2 * EMBED_DIM), jnp.float32),
            pltpu.VMEM((STRIPE, 2 * EMBED_DIM), jnp.float32),
            pltpu.VMEM((STRIPE, 2 * EMBED_DIM), jnp.float32),
            pltpu.VMEM((STRIPE, 2 * EMBED_DIM), jnp.float32),
            pltpu.VMEM((STRIPE,), jnp.float32),
            pltpu.VMEM((STRIPE,), jnp.float32),
            pltpu.SemaphoreType.DMA,
            pltpu.SemaphoreType.DMA,
            pltpu.SemaphoreType.DMA,
            pltpu.SemaphoreType.DMA,
            pltpu.SemaphoreType.DMA,
            pltpu.SemaphoreType.DMA,
        ],
    )
    def k(x2_hbm, emb2_hbm, out_hbm,
          idx0, idx1, sl0, sr0, sl1, sr1, hl0, hr0, hl1, hr1,
          rl0, rr0, rl1, rr1, ov0, ov1,
          semi0, semi1, semg0, semg1, semo0, semo1):
        wid = lax.axis_index("s") * 2 + lax.axis_index("c")
        lanes = lax.iota(jnp.int32, 16)

        idxs = (idx0, idx1)
        sups = ((sl0, sr0), (sl1, sr1))
        halfs = ((hl0, hr0), (hl1, hr1))
        rows = ((rl0, rr0), (rl1, rr1))
        semis = (semi0, semi1)
        semgs = (semg0, semg1)

        def fire_idx(a, par):
            pltpu.async_copy(
                x2_hbm.at[pl.ds((a * NW + wid) * 2, 2)], idxs[par],
                semis[par])

        def prep(a, par):
            idxb = idxs[par]
            supl, supr = sups[par]
            hlb, hrb = halfs[par]
            pltpu.make_async_copy(
                x2_hbm.at[pl.ds(0, 2)], idxb, semis[par]).wait()
            for kk in range(STRIPE // 16):
                sl16 = pl.ds(kk * 16, 16)
                vl = idxb[0, sl16]
                vr = idxb[1, sl16]
                supl[sl16] = lax.shift_right_logical(vl, 1)
                supr[sl16] = lax.shift_right_logical(vr, 1)
                hlb[sl16] = lax.bitwise_and(vl, 1) * EMBED_DIM
                hrb[sl16] = lax.bitwise_and(vr, 1) * EMBED_DIM

            @pl.when(a + 2 < NUM_AXIOMS)
            def _():
                fire_idx(a + 2, par)

            rl, rr = rows[par]
            pltpu.async_copy(emb2_hbm.at[supl], rl, semgs[par])
            pltpu.async_copy(emb2_hbm.at[supr], rr, semgs[par])

        def compute(a, par, ov, semo, first):
            rl, rr = rows[par]
            hlb, hrb = halfs[par]
            pltpu.make_async_copy(
                emb2_hbm.at[pl.ds(0, STRIPE)], rl, semgs[par]).wait()
            pltpu.make_async_copy(
                emb2_hbm.at[pl.ds(0, STRIPE)], rr, semgs[par]).wait()

            @pl.when(jnp.logical_not(first))
            def _():
                pltpu.make_async_copy(
                    out_hbm.at[0, pl.ds(0, STRIPE)], ov, semo).wait()

            def group(t, carry):
                e = 16 * t + lanes
                sl16 = pl.ds(16 * t, 16)
                half_l = hlb[sl16]
                half_r = hrb[sl16]
                base_col = lanes + t
                acc = jnp.zeros((16,), jnp.float32)
                for d in range(EMBED_DIM):
                    rot = lax.bitwise_and(base_col + d, 63)
                    vl = plsc.load_gather(rl, [e, half_l + rot])
                    vr = plsc.load_gather(rr, [e, half_r + rot])
                    acc = acc + vl * vr
                ov[sl16] = acc
                return carry

            lax.fori_loop(0, STRIPE // 16, group, 0)
            pltpu.async_copy(
                ov, out_hbm.at[a, pl.ds(wid * STRIPE, STRIPE)], semo)

        fire_idx(0, 0)
        fire_idx(1, 1)
        prep(0, 0)
        prep(1, 1)

        def body(i, carry):
            a0 = 2 * i
            compute(a0, 0, ov0, semo0, i == 0)

            @pl.when(a0 + 2 < NUM_AXIOMS)
            def _():
                prep(a0 + 2, 0)

            compute(a0 + 1, 1, ov1, semo1, i == 0)

            @pl.when(a0 + 3 < NUM_AXIOMS)
            def _():
                prep(a0 + 3, 1)

            return carry

        lax.fori_loop(0, NUM_AXIOMS // 2, body, 0)
        pltpu.make_async_copy(
            out_hbm.at[0, pl.ds(0, STRIPE)], ov0, semo0).wait()
        pltpu.make_async_copy(
            out_hbm.at[0, pl.ds(0, STRIPE)], ov1, semo1).wait()

    return k(x2, emb2)


def kernel(x, emb):
    embT = emb.T
    x2 = (x.astype(jnp.int32)
          .reshape(NW, STRIPE, NUM_AXIOMS, 2)
          .transpose(2, 0, 3, 1)
          .reshape(X2_ROWS, STRIPE))
    tail16k = emb[RT_TAIL_COL:].reshape(RT_TAIL_N // 2, 2 * EMBED_DIM)
    emb2 = _sc_retile(embT, tail16k)
    scores_t = _sc_score(x2, emb2)
    return scores_t.T
